# baseline (speedup 1.0000x reference)
import functools

import jax
import jax.numpy as jnp
import numpy as np
from jax.experimental import pallas as pl
from jax.experimental.pallas import tpu as pltpu

N = 10000
E = 320000
H = 128
L = 4
G = 500


def _out_proj_kernel(h_ref, w_ref, b_ref, o_ref):
    o_ref[...] = jax.nn.relu(
        jnp.dot(h_ref[...], w_ref[...], preferred_element_type=jnp.float32)
        + b_ref[...]
    )


def _out_proj(h, W_out, b_out):
    npad = ((N + 255) // 256) * 256
    hp = jnp.pad(h, ((0, npad - N), (0, 0)))
    out = pl.pallas_call(
        _out_proj_kernel,
        grid=(npad // 256,),
        in_specs=[
            pl.BlockSpec((256, H), lambda i: (i, 0)),
            pl.BlockSpec((H, H), lambda i: (0, 0)),
            pl.BlockSpec((H,), lambda i: (0,)),
        ],
        out_specs=pl.BlockSpec((256, H), lambda i: (i, 0)),
        out_shape=jax.ShapeDtypeStruct((npad, H), jnp.float32),
    )(hp, W_out, b_out)
    return out[:N]


def kernel(x, z, edge_index, bond_feature, edge_attr, peripheral_attr, rd, pos,
           batch, z_table, W_init, b_init, W_msg, W_edge, W_self, ln_g, ln_b,
           Wv1, bv1, Wv2, bv2, W_out, b_out):
    z_emb = jnp.take(z_table, z, axis=0)
    h = jnp.concatenate([z_emb, x], axis=-1) @ W_init + b_init
    src = edge_index[0]
    dst = edge_index[1]
    e = jnp.concatenate([bond_feature, edge_attr], axis=-1)
    vne = jnp.zeros((G, H), x.dtype)
    bn_scale = 1.0 / np.sqrt(1.0 + 1e-5)
    hcur = h
    for l in range(L):
        hl = hcur + jnp.take(vne, batch, axis=0)
        msg_src = jnp.take(hl @ W_msg[l], src, axis=0)
        m = jax.nn.relu(msg_src + e @ W_edge[l])
        agg = jax.ops.segment_sum(m, dst, num_segments=N)
        hn = agg + hl @ W_self[l]
        mu = jnp.mean(hn, axis=-1, keepdims=True)
        var = jnp.var(hn, axis=-1, keepdims=True)
        hn = (hn - mu) / jnp.sqrt(var + 1e-5) * ln_g[l] + ln_b[l]
        if l < L - 1:
            tmp = jax.ops.segment_sum(hl, batch, num_segments=G) + vne
            t = jax.nn.relu((tmp @ Wv1[l] + bv1[l]) * bn_scale)
            t = jax.nn.relu((t @ Wv2[l] + bv2[l]) * bn_scale)
            vne = t
        hcur = hn
    return _out_proj(hcur, W_out, b_out)


# speedup vs baseline: 2.7177x; 2.7177x over previous
import functools

import jax
import jax.numpy as jnp
import numpy as np
from jax import lax
from jax.experimental import pallas as pl
from jax.experimental.pallas import tpu as pltpu
from jax.experimental.pallas import tpu_sc as plsc

N = 10000
E = 320000
H = 128
L = 4
G = 500

NT = 16
NC = 2
C = 80
EPT = E // (NC * NT)
NCH = EPT // C
NP = 10240
RPT = NP // NT
ZR = 128


def _edge_body(a, ep, src, dst, out0, out1,
               agg_s, idx_s, idx_d, gbuf, pbuf, zbuf, sem):
    c = lax.axis_index("c")
    s = lax.axis_index("s")

    @pl.loop(0, ZR)
    def _zero(i):
        for k in range(8):
            zbuf[i, pl.ds(k * 16, 16)] = jnp.zeros((16,), jnp.float32)

    for i in range(RPT // ZR):
        pltpu.sync_copy(zbuf, agg_s.at[pl.ds(s * RPT + i * ZR, ZR)])
    plsc.subcore_barrier()

    @pl.loop(0, NCH)
    def _chunk(i):
        base = (c * NT + s) * EPT + i * C
        pltpu.sync_copy(src.at[pl.ds(base, C)], idx_s)
        pltpu.sync_copy(dst.at[pl.ds(base, C)], idx_d)
        pltpu.async_copy(a.at[idx_s], gbuf, sem).wait()
        pltpu.sync_copy(ep.at[pl.ds(base, C)], pbuf)

        @pl.loop(0, C)
        def _relu(j):
            for k in range(8):
                sl = pl.ds(k * 16, 16)
                pbuf[j, sl] = jnp.maximum(gbuf[j, sl] + pbuf[j, sl], 0.0)

        pltpu.sync_copy(pbuf, agg_s.at[idx_d], add=True)

    plsc.subcore_barrier()

    @pl.when(c == 0)
    def _o0():
        pltpu.sync_copy(agg_s.at[pl.ds(s * RPT, RPT)],
                        out0.at[pl.ds(s * RPT, RPT)])

    @pl.when(c == 1)
    def _o1():
        pltpu.sync_copy(agg_s.at[pl.ds(s * RPT, RPT)],
                        out1.at[pl.ds(s * RPT, RPT)])


@jax.jit
def _edge_sc(a, ep, src, dst):
    mesh = plsc.VectorSubcoreMesh(core_axis_name="c", subcore_axis_name="s")
    fn = pl.kernel(
        _edge_body,
        out_type=(jax.ShapeDtypeStruct((NP, H), jnp.float32),
                  jax.ShapeDtypeStruct((NP, H), jnp.float32)),
        mesh=mesh,
        scratch_types=[
            pltpu.VMEM_SHARED((NP, H), jnp.float32),
            pltpu.VMEM((C,), jnp.int32),
            pltpu.VMEM((C,), jnp.int32),
            pltpu.VMEM((C, H), jnp.float32),
            pltpu.VMEM((C, H), jnp.float32),
            pltpu.VMEM((ZR, H), jnp.float32),
            pltpu.SemaphoreType.DMA,
        ],
    )
    return fn(a, ep, src, dst)


def _out_proj_kernel(h_ref, w_ref, b_ref, o_ref):
    o_ref[...] = jax.nn.relu(
        jnp.dot(h_ref[...], w_ref[...], preferred_element_type=jnp.float32)
        + b_ref[...]
    )


def _out_proj(h, W_out, b_out):
    return pl.pallas_call(
        _out_proj_kernel,
        grid=(N // 400,),
        in_specs=[
            pl.BlockSpec((400, H), lambda i: (i, 0)),
            pl.BlockSpec((H, H), lambda i: (0, 0)),
            pl.BlockSpec((H,), lambda i: (0,)),
        ],
        out_specs=pl.BlockSpec((400, H), lambda i: (i, 0)),
        out_shape=jax.ShapeDtypeStruct((N, H), jnp.float32),
    )(h, W_out, b_out)


def kernel(x, z, edge_index, bond_feature, edge_attr, peripheral_attr, rd, pos,
           batch, z_table, W_init, b_init, W_msg, W_edge, W_self, ln_g, ln_b,
           Wv1, bv1, Wv2, bv2, W_out, b_out):
    z_emb = jnp.take(z_table, z, axis=0)
    h = jnp.concatenate([z_emb, x], axis=-1) @ W_init + b_init
    src = edge_index[0].astype(jnp.int32)
    dst = edge_index[1].astype(jnp.int32)
    e = jnp.concatenate([bond_feature, edge_attr], axis=-1)
    vne = jnp.zeros((G, H), x.dtype)
    bn_scale = 1.0 / np.sqrt(1.0 + 1e-5)
    hcur = h
    for l in range(L):
        hl = hcur + jnp.take(vne, batch, axis=0)
        A = hl @ W_msg[l]
        Ep = e @ W_edge[l]
        o0, o1 = _edge_sc(A, Ep, src, dst)
        agg = o0[:N] + o1[:N]
        hn = agg + hl @ W_self[l]
        mu = jnp.mean(hn, axis=-1, keepdims=True)
        var = jnp.var(hn, axis=-1, keepdims=True)
        hn = (hn - mu) / jnp.sqrt(var + 1e-5) * ln_g[l] + ln_b[l]
        if l < L - 1:
            tmp = jax.ops.segment_sum(hl, batch, num_segments=G) + vne
            t = jax.nn.relu((tmp @ Wv1[l] + bv1[l]) * bn_scale)
            t = jax.nn.relu((t @ Wv2[l] + bv2[l]) * bn_scale)
            vne = t
        hcur = hn
    return _out_proj(hcur, W_out, b_out)
